# Initial kernel scaffold; baseline (speedup 1.0000x reference)
import jax
import jax.numpy as jnp
from jax.experimental import pallas as pl


def kernel(x, edge_index, edge_type, basis1, comp1, q1, k1, bias1, skipW1, skipb1, gamma1, beta1, basis2, comp2, q2, k2, bias2, skipW2, skipb2):
    raise NotImplementedError("write your pallas kernel here")


# speedup vs baseline: 25.2474x; 25.2474x over previous
import functools

import jax
import jax.numpy as jnp
from jax import lax
from jax.experimental import pallas as pl
from jax.experimental.pallas import tpu as pltpu
from jax.experimental.pallas import tpu_sc as plsc

N = 10000
E = 160000
IN = 128
HID = 128
OUT = 64
R = 8
NB = 4
H = 2

NBLK = 10
BN_ = N // NBLK
NS = 16
EP = E // NS
CH = 80
NCH = EP // CH
NPAD = 10240
NROWT = NPAD // NS


def _dense_tables(x, basis, comp, q, k, out_ch, interpret=False):
    roww = out_ch + 16
    cin = x.shape[1]

    def body(x_ref, basis_ref, comp_ref, q_ref, k_ref, t_ref, aq_ref):
        r = pl.program_id(1)
        comp_r = comp_ref[pl.ds(r, 1), :][0]
        w = jnp.tensordot(comp_r, basis_ref[...], axes=1)
        y = jnp.dot(x_ref[...], w, preferred_element_type=jnp.float32)
        zeros15 = jnp.zeros((BN_, 15), jnp.float32)
        for h in range(H):
            y_h = y[:, h * out_ch:(h + 1) * out_ch]
            aq = jnp.dot(y_h, q_ref[h], preferred_element_type=jnp.float32)
            ak = jnp.dot(y_h, k_ref[h], preferred_element_type=jnp.float32)
            t_ref[h, 0] = jnp.concatenate([y_h, ak[:, None], zeros15], axis=1)
            aq_ref[h, 0] = jnp.concatenate([aq[:, None], zeros15], axis=1)

    t4, aq4 = pl.pallas_call(
        body,
        grid=(NBLK, R),
        in_specs=[
            pl.BlockSpec((BN_, cin), lambda b, r: (b, 0)),
            pl.BlockSpec((NB, cin, H * out_ch), lambda b, r: (0, 0, 0)),
            pl.BlockSpec((R, NB), lambda b, r: (0, 0)),
            pl.BlockSpec((H, out_ch), lambda b, r: (0, 0)),
            pl.BlockSpec((H, out_ch), lambda b, r: (0, 0)),
        ],
        out_specs=[
            pl.BlockSpec((H, 1, BN_, roww), lambda b, r: (0, r, b, 0)),
            pl.BlockSpec((H, 1, BN_, 16), lambda b, r: (0, r, b, 0)),
        ],
        out_shape=[
            jax.ShapeDtypeStruct((H, R, N, roww), jnp.float32),
            jax.ShapeDtypeStruct((H, R, N, 16), jnp.float32),
        ],
        interpret=interpret,
    )(x, basis, comp, q, k)
    return t4.reshape(H * R * N, roww), aq4.reshape(H * R * N, 16)


def _edge_phase(t_tab, aq_tab, gidx, didx, dst, zrow, out_ch, interpret=False):
    roww = out_ch + 16
    mesh = plsc.VectorSubcoreMesh(core_axis_name="c", subcore_axis_name="s",
                                  num_cores=H, num_subcores=NS)

    @functools.partial(
        pl.kernel,
        out_type=jax.ShapeDtypeStruct((H * NPAD, roww), jnp.float32),
        mesh=mesh,
        scratch_types=[
            pltpu.VMEM_SHARED((NPAD, roww), jnp.float32),
            pltpu.VMEM((CH,), jnp.int32),
            pltpu.VMEM((CH,), jnp.int32),
            pltpu.VMEM((CH,), jnp.int32),
            pltpu.VMEM((CH, roww), jnp.float32),
            pltpu.VMEM((CH, 16), jnp.float32),
            pltpu.SemaphoreType.DMA,
            pltpu.SemaphoreType.DMA,
        ],
        compiler_params=pltpu.CompilerParams(needs_layout_passes=False,
                                             use_tc_tiling_on_sc=False),
        interpret=interpret,
    )
    def k(t_hbm, aq_hbm, gidx_hbm, didx_hbm, dst_hbm, zrow_hbm, u_hbm,
          acc, gidx_v, didx_v, dst_v, a_v, qr_v, sem1, sem2):
        c = lax.axis_index("c")
        s = lax.axis_index("s")
        row0 = s * NROWT
        pltpu.sync_copy(zrow_hbm, acc.at[pl.ds(row0, NROWT)])
        plsc.subcore_barrier()

        ebase = s * EP
        lanes = lax.iota(jnp.int32, 16)

        def chunk(i, carry):
            base = ebase + i * CH
            pltpu.sync_copy(gidx_hbm.at[pl.ds(c * E + base, CH)], gidx_v)
            pltpu.sync_copy(didx_hbm.at[pl.ds(c * E + base, CH)], didx_v)
            pltpu.sync_copy(dst_hbm.at[pl.ds(base, CH)], dst_v)
            cp1 = pltpu.async_copy(t_hbm.at[gidx_v], a_v, sem1)
            cp2 = pltpu.async_copy(aq_hbm.at[didx_v], qr_v, sem2)
            cp1.wait()
            cp2.wait()
            for j in range(CH):
                tvec = qr_v[j, :] + a_v[j, pl.ds(out_ch, 16)]
                tsplat = plsc.cumsum(tvec)
                leak = jnp.maximum(tsplat, tsplat * 0.2)
                sj = jnp.exp(leak)
                for cb in range(out_ch // 16):
                    sl = pl.ds(cb * 16, 16)
                    a_v[j, sl] = a_v[j, sl] * sj
                a_v[j, pl.ds(out_ch, 16)] = jnp.where(lanes == 0, sj, 0.0)
            pltpu.sync_copy(a_v, acc.at[dst_v], add=True)
            return carry

        lax.fori_loop(0, NCH, chunk, 0)
        plsc.subcore_barrier()
        pltpu.sync_copy(acc.at[pl.ds(row0, NROWT)],
                        u_hbm.at[pl.ds(c * NPAD + row0, NROWT)])

    return k(t_tab, aq_tab, gidx, didx, dst, zrow)


def _post_layer1(u, x, skip_w, bias, skip_b, gamma, beta, interpret=False):
    o = HID

    def body(u_ref, x_ref, w_ref, b_ref, sb_ref, g_ref, be_ref, out_ref):
        u0 = u_ref[0]
        u1 = u_ref[1]
        m = (u0[:, :o] / (u0[:, o:o + 1] + 1e-16)
             + u1[:, :o] / (u1[:, o:o + 1] + 1e-16)) * 0.5
        g = (m + b_ref[0]
             + jnp.dot(x_ref[...], w_ref[...], preferred_element_type=jnp.float32)
             + sb_ref[0])
        mu = jnp.mean(g, axis=0, keepdims=True)
        xc = g - mu
        var = jnp.mean(xc * xc, axis=0, keepdims=True)
        out_ref[...] = jnp.maximum(
            xc / jnp.sqrt(var + 1e-5) * g_ref[0] + be_ref[0], 0.0)

    return pl.pallas_call(
        body,
        out_shape=jax.ShapeDtypeStruct((N, o), jnp.float32),
        interpret=interpret,
    )(u, x, skip_w, bias.reshape(1, o),
      skip_b.reshape(1, o), gamma.reshape(1, o), beta.reshape(1, o))


def _post_layer2(u, h_in, skip_w, bias, skip_b, interpret=False):
    o = OUT

    def body(u_ref, h_ref, w_ref, b_ref, sb_ref, out_ref):
        u0 = u_ref[0]
        u1 = u_ref[1]
        m = (u0[:, :o] / (u0[:, o:o + 1] + 1e-16)
             + u1[:, :o] / (u1[:, o:o + 1] + 1e-16)) * 0.5
        out_ref[...] = (m + b_ref[0]
                        + jnp.dot(h_ref[...], w_ref[...],
                                  preferred_element_type=jnp.float32)
                        + sb_ref[0])

    return pl.pallas_call(
        body,
        out_shape=jax.ShapeDtypeStruct((N, o), jnp.float32),
        interpret=interpret,
    )(u, h_in, skip_w, bias.reshape(1, o),
      skip_b.reshape(1, o))


def kernel(x, edge_index, edge_type, basis1, comp1, q1, k1, bias1, skipW1,
           skipb1, gamma1, beta1, basis2, comp2, q2, k2, bias2, skipW2,
           skipb2):
    src = edge_index[0]
    dst = edge_index[1]
    base_idx = edge_type * N
    gidx0 = base_idx + src
    didx0 = base_idx + dst
    gidx = jnp.concatenate([gidx0, gidx0 + R * N])
    didx = jnp.concatenate([didx0, didx0 + R * N])

    zrow1 = jnp.zeros((NROWT, HID + 16), jnp.float32)
    zrow2 = jnp.zeros((NROWT, OUT + 16), jnp.float32)

    t1, aq1 = _dense_tables(x, basis1, comp1, q1, k1, HID)
    u1 = _edge_phase(t1, aq1, gidx, didx, dst, zrow1, HID)
    h_ = _post_layer1(u1.reshape(H, NPAD, HID + 16)[:, :N], x, skipW1, bias1, skipb1, gamma1, beta1)

    t2, aq2 = _dense_tables(h_, basis2, comp2, q2, k2, OUT)
    u2 = _edge_phase(t2, aq2, gidx, didx, dst, zrow2, OUT)
    return _post_layer2(u2.reshape(H, NPAD, OUT + 16)[:, :N], h_, skipW2, bias2, skipb2)


# speedup vs baseline: 29.4525x; 1.1666x over previous
import functools

import jax
import jax.numpy as jnp
from jax import lax
from jax.experimental import pallas as pl
from jax.experimental.pallas import tpu as pltpu
from jax.experimental.pallas import tpu_sc as plsc

N = 10000
E = 160000
IN = 128
HID = 128
OUT = 64
R = 8
NB = 4
H = 2

NBLK = 10
BN_ = N // NBLK
NS = 16
EP = E // NS
CH = 40
NCH = EP // CH
NPAIR = NCH // 2
NPAD = 10240
NROWT = NPAD // NS
ZR = 64


def _dense_tables(x, basis4, comp, q3, k3, out_ch, interpret=False):
    roww = out_ch + 16
    cin = x.shape[1]
    WP = 128

    def body(x_ref, basis_ref, comp_ref, q_ref, k_ref, t_ref, aq_ref):
        r = pl.program_id(2)
        comp_r = comp_ref[pl.ds(r, 1), :][0]
        w = jnp.tensordot(comp_r, basis_ref[0], axes=1)
        y = jnp.dot(x_ref[...], w, preferred_element_type=jnp.float32)
        aq = jnp.dot(y, q_ref[0, 0], preferred_element_type=jnp.float32)
        ak = jnp.dot(y, k_ref[0, 0], preferred_element_type=jnp.float32)
        zeros15 = jnp.zeros((BN_, 15), jnp.float32)
        t_ref[...] = jnp.concatenate([y[:, :out_ch], ak[:, None], zeros15],
                                     axis=1)
        aq_ref[...] = jnp.concatenate([aq[:, None], zeros15], axis=1)

    return pl.pallas_call(
        body,
        grid=(H, NBLK, R),
        in_specs=[
            pl.BlockSpec((BN_, cin), lambda h, b, r: (b, 0)),
            pl.BlockSpec((1, NB, cin, WP), lambda h, b, r: (h, 0, 0, 0)),
            pl.BlockSpec((R, NB), lambda h, b, r: (0, 0)),
            pl.BlockSpec((1, 1, WP), lambda h, b, r: (h, 0, 0)),
            pl.BlockSpec((1, 1, WP), lambda h, b, r: (h, 0, 0)),
        ],
        out_specs=[
            pl.BlockSpec((BN_, roww),
                         lambda h, b, r: (h * R * NBLK + r * NBLK + b, 0)),
            pl.BlockSpec((BN_, 16),
                         lambda h, b, r: (h * R * NBLK + r * NBLK + b, 0)),
        ],
        out_shape=[
            jax.ShapeDtypeStruct((H * R * N, roww), jnp.float32),
            jax.ShapeDtypeStruct((H * R * N, 16), jnp.float32),
        ],
        interpret=interpret,
    )(x, basis4, comp, q3, k3)


def _edge_phase(t_tab, aq_tab, cat, dst3, zrow, out_ch, interpret=False):
    roww = out_ch + 16
    mesh = plsc.VectorSubcoreMesh(core_axis_name="c", subcore_axis_name="s",
                                  num_cores=H, num_subcores=NS)

    @functools.partial(
        pl.kernel,
        out_type=jax.ShapeDtypeStruct((H, NPAD, roww), jnp.float32),
        mesh=mesh,
        scratch_types=[
            pltpu.VMEM_SHARED((NPAD, roww), jnp.float32),
            pltpu.VMEM((NCH, CH), jnp.int32),
            pltpu.VMEM((2, 2, CH), jnp.int32),
            pltpu.VMEM((2, CH, roww), jnp.float32),
            pltpu.VMEM((2, CH, 16), jnp.float32),
            pltpu.VMEM((2, CH, roww), jnp.float32),
            pltpu.SemaphoreType.DMA,
            pltpu.SemaphoreType.DMA,
            pltpu.SemaphoreType.DMA,
            pltpu.SemaphoreType.DMA,
            pltpu.SemaphoreType.DMA,
            pltpu.SemaphoreType.DMA,
        ],
        compiler_params=pltpu.CompilerParams(needs_layout_passes=False,
                                             use_tc_tiling_on_sc=False),
        interpret=interpret,
    )
    def k(t_hbm, aq_hbm, cat_hbm, dst_hbm, zrow_hbm, u_hbm,
          acc, dst_v, idx_v, a_v, qr_v, b_v,
          semg0, semg1, semi0, semi1, sems0, sems1):
        c = lax.axis_index("c")
        s = lax.axis_index("s")
        semg = (semg0, semg1)
        semi = (semi0, semi1)
        sems = (sems0, sems1)
        row0 = s * NROWT
        lanes = lax.iota(jnp.int32, 16)

        def issue_gather(g, b):
            pltpu.async_copy(t_hbm.at[idx_v.at[b, 0]], a_v.at[b], semg[b])
            pltpu.async_copy(aq_hbm.at[idx_v.at[b, 1]], qr_v.at[b], semg[b])

        def wait_gather(g, b):
            pltpu.make_async_copy(t_hbm.at[idx_v.at[b, 0]], a_v.at[b],
                                  semg[b]).wait()
            pltpu.make_async_copy(aq_hbm.at[idx_v.at[b, 1]], qr_v.at[b],
                                  semg[b]).wait()

        def issue_idx(g, b):
            pltpu.async_copy(cat_hbm.at[c, s, g], idx_v.at[b], semi[b])

        def wait_idx(g, b):
            pltpu.make_async_copy(cat_hbm.at[c, s, g], idx_v.at[b],
                                  semi[b]).wait()

        def issue_scatter(g, b):
            pltpu.async_copy(b_v.at[b], acc.at[dst_v.at[g]], sems[b], add=True)

        def wait_scatter(g, b):
            pltpu.make_async_copy(b_v.at[b], acc.at[dst_v.at[g]],
                                  sems[b]).wait()

        pltpu.sync_copy(dst_hbm.at[s], dst_v)
        pltpu.sync_copy(cat_hbm.at[c, s, 0], idx_v.at[0])
        issue_idx(1, 1)
        issue_gather(0, 0)
        def zstripe(z, carry):
            pltpu.sync_copy(zrow_hbm, acc.at[pl.ds(row0 + z * ZR, ZR)])
            return carry
        lax.fori_loop(0, NROWT // ZR, zstripe, 0)
        plsc.subcore_barrier()

        def pair(p, carry):
            for b in range(2):
                g = 2 * p + b
                wait_gather(g, b)

                @pl.when(g >= 2)
                def _():
                    wait_scatter(g - 2, b)

                for j in range(CH):
                    tvec = qr_v[b, j, :] + a_v[b, j, pl.ds(out_ch, 16)]
                    tsplat = plsc.cumsum(tvec)
                    leak = jnp.maximum(tsplat, tsplat * 0.2)
                    sj = jnp.exp(leak)
                    for cb in range(out_ch // 16):
                        sl = pl.ds(cb * 16, 16)
                        b_v[b, j, sl] = a_v[b, j, sl] * sj
                    b_v[b, j, pl.ds(out_ch, 16)] = jnp.where(lanes == 0, sj, 0.0)
                issue_scatter(g, b)

                @pl.when(g + 1 < NCH)
                def _():
                    wait_idx(g + 1, 1 - b)
                    issue_gather(g + 1, 1 - b)

                @pl.when(g + 2 < NCH)
                def _():
                    issue_idx(g + 2, b)
            return carry

        lax.fori_loop(0, NPAIR, pair, 0)
        wait_scatter(NCH - 2, 0)
        wait_scatter(NCH - 1, 1)
        plsc.subcore_barrier()
        pltpu.sync_copy(acc.at[pl.ds(row0, NROWT)],
                        u_hbm.at[c, pl.ds(row0, NROWT)])

    return k(t_tab, aq_tab, cat, dst3, zrow)


def _post_layer1(u, x, skip_w, bias, skip_b, gamma, beta, interpret=False):
    o = HID

    def body(u_ref, x_ref, w_ref, b_ref, sb_ref, g_ref, be_ref, out_ref):
        u0 = u_ref[0, 0:N]
        u1 = u_ref[1, 0:N]
        m = (u0[:, :o] / (u0[:, o:o + 1] + 1e-16)
             + u1[:, :o] / (u1[:, o:o + 1] + 1e-16)) * 0.5
        g = (m + b_ref[0]
             + jnp.dot(x_ref[...], w_ref[...], preferred_element_type=jnp.float32)
             + sb_ref[0])
        mu = jnp.mean(g, axis=0, keepdims=True)
        xc = g - mu
        var = jnp.mean(xc * xc, axis=0, keepdims=True)
        out_ref[...] = jnp.maximum(
            xc / jnp.sqrt(var + 1e-5) * g_ref[0] + be_ref[0], 0.0)

    return pl.pallas_call(
        body,
        out_shape=jax.ShapeDtypeStruct((N, o), jnp.float32),
        interpret=interpret,
    )(u, x, skip_w, bias.reshape(1, o),
      skip_b.reshape(1, o), gamma.reshape(1, o), beta.reshape(1, o))


def _post_layer2(u, h_in, skip_w, bias, skip_b, interpret=False):
    o = OUT

    def body(u_ref, h_ref, w_ref, b_ref, sb_ref, out_ref):
        u0 = u_ref[0, 0:N]
        u1 = u_ref[1, 0:N]
        m = (u0[:, :o] / (u0[:, o:o + 1] + 1e-16)
             + u1[:, :o] / (u1[:, o:o + 1] + 1e-16)) * 0.5
        out_ref[...] = (m + b_ref[0]
                        + jnp.dot(h_ref[...], w_ref[...],
                                  preferred_element_type=jnp.float32)
                        + sb_ref[0])

    return pl.pallas_call(
        body,
        out_shape=jax.ShapeDtypeStruct((N, o), jnp.float32),
        interpret=interpret,
    )(u, h_in, skip_w, bias.reshape(1, o), skip_b.reshape(1, o))


def _head_major(basis, q, k, cin, out_ch):
    wp = 128
    b4 = basis.reshape(NB, cin, H, out_ch).transpose(2, 0, 1, 3)
    pad = ((0, 0), (0, 0), (0, 0), (0, wp - out_ch))
    b4 = jnp.pad(b4, pad)
    q3 = jnp.pad(q.reshape(H, 1, out_ch), ((0, 0), (0, 0), (0, wp - out_ch)))
    k3 = jnp.pad(k.reshape(H, 1, out_ch), ((0, 0), (0, 0), (0, wp - out_ch)))
    return b4, q3, k3


def kernel(x, edge_index, edge_type, basis1, comp1, q1, k1, bias1, skipW1,
           skipb1, gamma1, beta1, basis2, comp2, q2, k2, bias2, skipW2,
           skipb2):
    src = edge_index[0]
    dst = edge_index[1]
    base_idx = edge_type * N
    g0 = (base_idx + src).reshape(NS, NCH, CH)
    d0 = (base_idx + dst).reshape(NS, NCH, CH)
    cat = jnp.stack([jnp.stack([g0, d0], axis=2),
                     jnp.stack([g0 + R * N, d0 + R * N], axis=2)])
    dst3 = dst.reshape(NS, NCH, CH)

    zrow1 = jnp.zeros((ZR, HID + 16), jnp.float32)
    zrow2 = jnp.zeros((ZR, OUT + 16), jnp.float32)

    b41, q31, k31 = _head_major(basis1, q1, k1, IN, HID)
    t1, aq1 = _dense_tables(x, b41, comp1, q31, k31, HID)
    u1 = _edge_phase(t1, aq1, cat, dst3, zrow1, HID)
    h_ = _post_layer1(u1, x, skipW1, bias1, skipb1, gamma1, beta1)

    b42, q32, k32 = _head_major(basis2, q2, k2, HID, OUT)
    t2, aq2 = _dense_tables(h_, b42, comp2, q32, k32, OUT)
    u2 = _edge_phase(t2, aq2, cat, dst3, zrow2, OUT)
    return _post_layer2(u2, h_, skipW2, bias2, skipb2)


# speedup vs baseline: 30.8833x; 1.0486x over previous
import functools

import jax
import jax.numpy as jnp
from jax import lax
from jax.experimental import pallas as pl
from jax.experimental.pallas import tpu as pltpu
from jax.experimental.pallas import tpu_sc as plsc

N = 10000
E = 160000
IN = 128
HID = 128
OUT = 64
R = 8
NB = 4
H = 2

NBLK = 10
BN_ = N // NBLK
NS = 16
EP = E // NS
CH = 40
NCH = EP // CH
NPAIR = NCH // 2
NPAD = 10240
NROWT = NPAD // NS
ZR = 64


def _dense_tables(x, basis4, comp, q3, k3, out_ch, roww_t, interpret=False):
    roww = roww_t
    cin = x.shape[1]
    WP = 128

    def body(x_ref, basis_ref, comp_ref, q_ref, k_ref, t_ref, aq_ref):
        r = pl.program_id(2)
        comp_r = comp_ref[pl.ds(r, 1), :][0]
        w = jnp.tensordot(comp_r, basis_ref[0], axes=1)
        y = jnp.dot(x_ref[...], w, preferred_element_type=jnp.float32)
        aq = jnp.dot(y, q_ref[0, 0], preferred_element_type=jnp.float32)
        ak = jnp.dot(y, k_ref[0, 0], preferred_element_type=jnp.float32)
        zeros15 = jnp.zeros((BN_, 15), jnp.float32)
        ztail = jnp.zeros((BN_, roww_t - out_ch - 1), jnp.float32)
        t_ref[...] = jnp.concatenate([y[:, :out_ch], ak[:, None], ztail],
                                     axis=1)
        aq_ref[...] = jnp.concatenate([aq[:, None], zeros15], axis=1)

    return pl.pallas_call(
        body,
        grid=(H, NBLK, R),
        in_specs=[
            pl.BlockSpec((BN_, cin), lambda h, b, r: (b, 0)),
            pl.BlockSpec((1, NB, cin, WP), lambda h, b, r: (h, 0, 0, 0)),
            pl.BlockSpec((R, NB), lambda h, b, r: (0, 0)),
            pl.BlockSpec((1, 1, WP), lambda h, b, r: (h, 0, 0)),
            pl.BlockSpec((1, 1, WP), lambda h, b, r: (h, 0, 0)),
        ],
        out_specs=[
            pl.BlockSpec((BN_, roww),
                         lambda h, b, r: (h * R * NBLK + r * NBLK + b, 0)),
            pl.BlockSpec((BN_, 16),
                         lambda h, b, r: (h * R * NBLK + r * NBLK + b, 0)),
        ],
        out_shape=[
            jax.ShapeDtypeStruct((H * R * N, roww), jnp.float32),
            jax.ShapeDtypeStruct((H * R * N, 16), jnp.float32),
        ],
        interpret=interpret,
    )(x, basis4, comp, q3, k3)


def _edge_phase(t_tab, aq_tab, cat, dst3, zrow, out_ch, roww_t, interpret=False):
    roww = out_ch + 16
    mesh = plsc.VectorSubcoreMesh(core_axis_name="c", subcore_axis_name="s",
                                  num_cores=H, num_subcores=NS)

    @functools.partial(
        pl.kernel,
        out_type=jax.ShapeDtypeStruct((H, NPAD, roww), jnp.float32),
        mesh=mesh,
        scratch_types=[
            pltpu.VMEM_SHARED((NPAD, roww), jnp.float32),
            pltpu.VMEM((NCH, CH), jnp.int32),
            pltpu.VMEM((2, 2, CH), jnp.int32),
            pltpu.VMEM((2, CH, roww_t), jnp.float32),
            pltpu.VMEM((2, CH, 16), jnp.float32),
            pltpu.VMEM((2, CH, roww), jnp.float32),
            pltpu.SemaphoreType.DMA,
            pltpu.SemaphoreType.DMA,
            pltpu.SemaphoreType.DMA,
            pltpu.SemaphoreType.DMA,
            pltpu.SemaphoreType.DMA,
            pltpu.SemaphoreType.DMA,
        ],
        compiler_params=pltpu.CompilerParams(needs_layout_passes=False,
                                             use_tc_tiling_on_sc=False),
        interpret=interpret,
    )
    def k(t_hbm, aq_hbm, cat_hbm, dst_hbm, zrow_hbm, u_hbm,
          acc, dst_v, idx_v, a_v, qr_v, b_v,
          semg0, semg1, semi0, semi1, sems0, sems1):
        c = lax.axis_index("c")
        s = lax.axis_index("s")
        semg = (semg0, semg1)
        semi = (semi0, semi1)
        sems = (sems0, sems1)
        row0 = s * NROWT
        lanes = lax.iota(jnp.int32, 16)

        def issue_gather(g, b):
            pltpu.async_copy(t_hbm.at[idx_v.at[b, 0]], a_v.at[b], semg[b])
            pltpu.async_copy(aq_hbm.at[idx_v.at[b, 1]], qr_v.at[b], semg[b])

        def wait_gather(g, b):
            pltpu.make_async_copy(t_hbm.at[idx_v.at[b, 0]], a_v.at[b],
                                  semg[b]).wait()
            pltpu.make_async_copy(aq_hbm.at[idx_v.at[b, 1]], qr_v.at[b],
                                  semg[b]).wait()

        def issue_idx(g, b):
            pltpu.async_copy(cat_hbm.at[c, s, g], idx_v.at[b], semi[b])

        def wait_idx(g, b):
            pltpu.make_async_copy(cat_hbm.at[c, s, g], idx_v.at[b],
                                  semi[b]).wait()

        def issue_scatter(g, b):
            pltpu.async_copy(b_v.at[b], acc.at[dst_v.at[g]], sems[b], add=True)

        def wait_scatter(g, b):
            pltpu.make_async_copy(b_v.at[b], acc.at[dst_v.at[g]],
                                  sems[b]).wait()

        pltpu.sync_copy(dst_hbm.at[s], dst_v)
        pltpu.sync_copy(cat_hbm.at[c, s, 0], idx_v.at[0])
        issue_idx(1, 1)
        issue_gather(0, 0)
        def zstripe(z, carry):
            pltpu.sync_copy(zrow_hbm, acc.at[pl.ds(row0 + z * ZR, ZR)])
            return carry
        lax.fori_loop(0, NROWT // ZR, zstripe, 0)
        plsc.subcore_barrier()

        def pair(p, carry):
            for b in range(2):
                g = 2 * p + b
                wait_gather(g, b)

                @pl.when(g >= 2)
                def _():
                    wait_scatter(g - 2, b)

                for j in range(CH):
                    tvec = qr_v[b, j, :] + a_v[b, j, pl.ds(out_ch, 16)]
                    tsplat = plsc.cumsum(tvec)
                    leak = jnp.maximum(tsplat, tsplat * 0.2)
                    sj = jnp.exp(leak)
                    for cb in range(out_ch // 16):
                        sl = pl.ds(cb * 16, 16)
                        b_v[b, j, sl] = a_v[b, j, sl] * sj
                    b_v[b, j, pl.ds(out_ch, 16)] = jnp.where(lanes == 0, sj, 0.0)
                issue_scatter(g, b)

                @pl.when(g + 1 < NCH)
                def _():
                    wait_idx(g + 1, 1 - b)
                    issue_gather(g + 1, 1 - b)

                @pl.when(g + 2 < NCH)
                def _():
                    issue_idx(g + 2, b)
            return carry

        lax.fori_loop(0, NPAIR, pair, 0)
        wait_scatter(NCH - 2, 0)
        wait_scatter(NCH - 1, 1)
        plsc.subcore_barrier()
        pltpu.sync_copy(acc.at[pl.ds(row0, NROWT)],
                        u_hbm.at[c, pl.ds(row0, NROWT)])

    return k(t_tab, aq_tab, cat, dst3, zrow)


def _post_layer1(u, x, skip_w, bias, skip_b, gamma, beta, interpret=False):
    o = HID

    def body(u_ref, x_ref, w_ref, b_ref, sb_ref, g_ref, be_ref, out_ref):
        u0 = u_ref[0, 0:N]
        u1 = u_ref[1, 0:N]
        m = (u0[:, :o] / (u0[:, o:o + 1] + 1e-16)
             + u1[:, :o] / (u1[:, o:o + 1] + 1e-16)) * 0.5
        g = (m + b_ref[0]
             + jnp.dot(x_ref[...], w_ref[...], preferred_element_type=jnp.float32)
             + sb_ref[0])
        mu = jnp.mean(g, axis=0, keepdims=True)
        xc = g - mu
        var = jnp.mean(xc * xc, axis=0, keepdims=True)
        out_ref[...] = jnp.maximum(
            xc / jnp.sqrt(var + 1e-5) * g_ref[0] + be_ref[0], 0.0)

    return pl.pallas_call(
        body,
        out_shape=jax.ShapeDtypeStruct((N, o), jnp.float32),
        interpret=interpret,
    )(u, x, skip_w, bias.reshape(1, o),
      skip_b.reshape(1, o), gamma.reshape(1, o), beta.reshape(1, o))


def _post_layer2(u, h_in, skip_w, bias, skip_b, interpret=False):
    o = OUT

    def body(u_ref, h_ref, w_ref, b_ref, sb_ref, out_ref):
        u0 = u_ref[0, 0:N]
        u1 = u_ref[1, 0:N]
        m = (u0[:, :o] / (u0[:, o:o + 1] + 1e-16)
             + u1[:, :o] / (u1[:, o:o + 1] + 1e-16)) * 0.5
        out_ref[...] = (m + b_ref[0]
                        + jnp.dot(h_ref[...], w_ref[...],
                                  preferred_element_type=jnp.float32)
                        + sb_ref[0])

    return pl.pallas_call(
        body,
        out_shape=jax.ShapeDtypeStruct((N, o), jnp.float32),
        interpret=interpret,
    )(u, h_in, skip_w, bias.reshape(1, o), skip_b.reshape(1, o))


def _head_major(basis, q, k, cin, out_ch):
    wp = 128
    b4 = basis.reshape(NB, cin, H, out_ch).transpose(2, 0, 1, 3)
    pad = ((0, 0), (0, 0), (0, 0), (0, wp - out_ch))
    b4 = jnp.pad(b4, pad)
    q3 = jnp.pad(q.reshape(H, 1, out_ch), ((0, 0), (0, 0), (0, wp - out_ch)))
    k3 = jnp.pad(k.reshape(H, 1, out_ch), ((0, 0), (0, 0), (0, wp - out_ch)))
    return b4, q3, k3


def kernel(x, edge_index, edge_type, basis1, comp1, q1, k1, bias1, skipW1,
           skipb1, gamma1, beta1, basis2, comp2, q2, k2, bias2, skipW2,
           skipb2):
    src = edge_index[0]
    dst = edge_index[1]
    base_idx = edge_type * N
    g0 = (base_idx + src).reshape(NS, NCH, CH)
    d0 = (base_idx + dst).reshape(NS, NCH, CH)
    cat = jnp.stack([jnp.stack([g0, d0], axis=2),
                     jnp.stack([g0 + R * N, d0 + R * N], axis=2)])
    dst3 = dst.reshape(NS, NCH, CH)

    zrow1 = jnp.zeros((ZR, HID + 16), jnp.float32)
    zrow2 = jnp.zeros((ZR, OUT + 16), jnp.float32)

    b41, q31, k31 = _head_major(basis1, q1, k1, IN, HID)
    t1, aq1 = _dense_tables(x, b41, comp1, q31, k31, HID, HID + 16)
    u1 = _edge_phase(t1, aq1, cat, dst3, zrow1, HID, HID + 16)
    h_ = _post_layer1(u1, x, skipW1, bias1, skipb1, gamma1, beta1)

    b42, q32, k32 = _head_major(basis2, q2, k2, HID, OUT)
    t2, aq2 = _dense_tables(h_, b42, comp2, q32, k32, OUT, 128)
    u2 = _edge_phase(t2, aq2, cat, dst3, zrow2, OUT, 128)
    return _post_layer2(u2, h_, skipW2, bias2, skipb2)


# speedup vs baseline: 34.9527x; 1.1318x over previous
import functools

import jax
import jax.numpy as jnp
from jax import lax
from jax.experimental import pallas as pl
from jax.experimental.pallas import tpu as pltpu
from jax.experimental.pallas import tpu_sc as plsc

N = 10000
E = 160000
IN = 128
HID = 128
OUT = 64
R = 8
NB = 4
H = 2

NBLK = 10
BN_ = N // NBLK
NS = 16
EP = E // NS
CH = 40
NCH = EP // CH
NPAIR = NCH // 2
NPAD = 10240
NROWT = NPAD // NS
ZR = 64


def _dense_tables(x, basis4, comp, q3, k3, out_ch, interpret=False):
    cin = x.shape[1]
    WP = 128

    def body(x_ref, basis_ref, comp_ref, q_ref, k_ref, t_ref, q2_ref):
        r = pl.program_id(2)
        comp_r = comp_ref[pl.ds(r, 1), :][0]
        w = jnp.tensordot(comp_r, basis_ref[0], axes=1)
        y = jnp.dot(x_ref[...], w, preferred_element_type=jnp.float32)
        aq = jnp.dot(y, q_ref[0, 0], preferred_element_type=jnp.float32)
        ak = jnp.dot(y, k_ref[0, 0], preferred_element_type=jnp.float32)
        zeros15 = jnp.zeros((BN_, 15), jnp.float32)
        if out_ch == WP:
            t_ref[...] = y
        else:
            ztail = jnp.zeros((BN_, WP - out_ch - 1), jnp.float32)
            t_ref[...] = jnp.concatenate([y[:, :out_ch], ak[:, None], ztail],
                                         axis=1)
        q2_ref[...] = jnp.concatenate([aq[:, None], zeros15,
                                       ak[:, None], zeros15], axis=1)

    return pl.pallas_call(
        body,
        grid=(H, NBLK, R),
        in_specs=[
            pl.BlockSpec((BN_, cin), lambda h, b, r: (b, 0)),
            pl.BlockSpec((1, NB, cin, WP), lambda h, b, r: (h, 0, 0, 0)),
            pl.BlockSpec((R, NB), lambda h, b, r: (0, 0)),
            pl.BlockSpec((1, 1, WP), lambda h, b, r: (h, 0, 0)),
            pl.BlockSpec((1, 1, WP), lambda h, b, r: (h, 0, 0)),
        ],
        out_specs=[
            pl.BlockSpec((BN_, WP),
                         lambda h, b, r: (h * R * NBLK + r * NBLK + b, 0)),
            pl.BlockSpec((BN_, 32),
                         lambda h, b, r: (h * R * NBLK + r * NBLK + b, 0)),
        ],
        out_shape=[
            jax.ShapeDtypeStruct((H * R * N, WP), jnp.float32),
            jax.ShapeDtypeStruct((H * R * N, 32), jnp.float32),
        ],
        interpret=interpret,
    )(x, basis4, comp, q3, k3)


def _edge_phase(t_tab, q_tab, cat, dst3, zrow, out_ch, qs, interpret=False):
    accw = out_ch + 16
    mesh = plsc.VectorSubcoreMesh(core_axis_name="c", subcore_axis_name="s",
                                  num_cores=H, num_subcores=NS)

    @functools.partial(
        pl.kernel,
        out_type=jax.ShapeDtypeStruct((H, NPAD, accw), jnp.float32),
        mesh=mesh,
        scratch_types=[
            pltpu.VMEM_SHARED((NPAD, accw), jnp.float32),
            pltpu.VMEM((NCH, CH), jnp.int32),
            pltpu.VMEM((2, 2, CH), jnp.int32),
            pltpu.VMEM((2, CH, 128), jnp.float32),
            pltpu.VMEM((2, CH, 32), jnp.float32),
            pltpu.VMEM((2, CH, 32), jnp.float32),
            pltpu.VMEM((2, CH, accw), jnp.float32),
            pltpu.SemaphoreType.DMA,
            pltpu.SemaphoreType.DMA,
            pltpu.SemaphoreType.DMA,
            pltpu.SemaphoreType.DMA,
            pltpu.SemaphoreType.DMA,
            pltpu.SemaphoreType.DMA,
        ],
        compiler_params=pltpu.CompilerParams(needs_layout_passes=False,
                                             use_tc_tiling_on_sc=False),
        interpret=interpret,
    )
    def k(t_hbm, q_hbm, cat_hbm, dst_hbm, zrow_hbm, u_hbm,
          acc, dst_v, idx_v, a_v, qd_v, qs_v, b_v,
          semg0, semg1, semi0, semi1, sems0, sems1):
        c = lax.axis_index("c")
        s = lax.axis_index("s")
        semg = (semg0, semg1)
        semi = (semi0, semi1)
        sems = (sems0, sems1)
        row0 = s * NROWT
        lanes = lax.iota(jnp.int32, 16)

        def issue_gather(g, b):
            pltpu.async_copy(t_hbm.at[idx_v.at[b, 0]], a_v.at[b], semg[b])
            pltpu.async_copy(q_hbm.at[idx_v.at[b, 1]], qd_v.at[b], semg[b])
            if qs:
                pltpu.async_copy(q_hbm.at[idx_v.at[b, 0]], qs_v.at[b], semg[b])

        def wait_gather(g, b):
            pltpu.make_async_copy(t_hbm.at[idx_v.at[b, 0]], a_v.at[b],
                                  semg[b]).wait()
            pltpu.make_async_copy(q_hbm.at[idx_v.at[b, 1]], qd_v.at[b],
                                  semg[b]).wait()
            if qs:
                pltpu.make_async_copy(q_hbm.at[idx_v.at[b, 0]], qs_v.at[b],
                                      semg[b]).wait()

        def issue_idx(g, b):
            pltpu.async_copy(cat_hbm.at[c, s, g], idx_v.at[b], semi[b])

        def wait_idx(g, b):
            pltpu.make_async_copy(cat_hbm.at[c, s, g], idx_v.at[b],
                                  semi[b]).wait()

        def issue_scatter(g, b):
            pltpu.async_copy(b_v.at[b], acc.at[dst_v.at[g]], sems[b], add=True)

        def wait_scatter(g, b):
            pltpu.make_async_copy(b_v.at[b], acc.at[dst_v.at[g]],
                                  sems[b]).wait()

        pltpu.sync_copy(dst_hbm.at[s], dst_v)
        pltpu.sync_copy(cat_hbm.at[c, s, 0], idx_v.at[0])
        issue_idx(1, 1)
        issue_gather(0, 0)
        def zstripe(z, carry):
            pltpu.sync_copy(zrow_hbm, acc.at[pl.ds(row0 + z * ZR, ZR)])
            return carry
        lax.fori_loop(0, NROWT // ZR, zstripe, 0)
        plsc.subcore_barrier()

        def pair(p, carry):
            for b in range(2):
                g = 2 * p + b
                wait_gather(g, b)

                @pl.when(g >= 2)
                def _():
                    wait_scatter(g - 2, b)

                for j in range(CH):
                    if qs:
                        tvec = qd_v[b, j, pl.ds(0, 16)] + qs_v[b, j, pl.ds(16, 16)]
                    else:
                        tvec = qd_v[b, j, pl.ds(0, 16)] + a_v[b, j, pl.ds(out_ch, 16)]
                    tsplat = plsc.cumsum(tvec)
                    leak = jnp.maximum(tsplat, tsplat * 0.2)
                    sj = jnp.exp(leak)
                    for cb in range(out_ch // 16):
                        sl = pl.ds(cb * 16, 16)
                        b_v[b, j, sl] = a_v[b, j, sl] * sj
                    b_v[b, j, pl.ds(out_ch, 16)] = jnp.where(lanes == 0, sj, 0.0)
                issue_scatter(g, b)

                @pl.when(g + 1 < NCH)
                def _():
                    wait_idx(g + 1, 1 - b)
                    issue_gather(g + 1, 1 - b)

                @pl.when(g + 2 < NCH)
                def _():
                    issue_idx(g + 2, b)
            return carry

        lax.fori_loop(0, NPAIR, pair, 0)
        wait_scatter(NCH - 2, 0)
        wait_scatter(NCH - 1, 1)
        plsc.subcore_barrier()
        pltpu.sync_copy(acc.at[pl.ds(row0, NROWT)],
                        u_hbm.at[c, pl.ds(row0, NROWT)])

    return k(t_tab, q_tab, cat, dst3, zrow)


def _post_layer1(u, x, skip_w, bias, skip_b, gamma, beta, interpret=False):
    o = HID

    def body(u_ref, x_ref, w_ref, b_ref, sb_ref, g_ref, be_ref, out_ref):
        u0 = u_ref[0, 0:N]
        u1 = u_ref[1, 0:N]
        m = (u0[:, :o] / (u0[:, o:o + 1] + 1e-16)
             + u1[:, :o] / (u1[:, o:o + 1] + 1e-16)) * 0.5
        g = (m + b_ref[0]
             + jnp.dot(x_ref[...], w_ref[...], preferred_element_type=jnp.float32)
             + sb_ref[0])
        mu = jnp.mean(g, axis=0, keepdims=True)
        xc = g - mu
        var = jnp.mean(xc * xc, axis=0, keepdims=True)
        out_ref[...] = jnp.maximum(
            xc / jnp.sqrt(var + 1e-5) * g_ref[0] + be_ref[0], 0.0)

    return pl.pallas_call(
        body,
        out_shape=jax.ShapeDtypeStruct((N, o), jnp.float32),
        interpret=interpret,
    )(u, x, skip_w, bias.reshape(1, o),
      skip_b.reshape(1, o), gamma.reshape(1, o), beta.reshape(1, o))


def _post_layer2(u, h_in, skip_w, bias, skip_b, interpret=False):
    o = OUT

    def body(u_ref, h_ref, w_ref, b_ref, sb_ref, out_ref):
        u0 = u_ref[0, 0:N]
        u1 = u_ref[1, 0:N]
        m = (u0[:, :o] / (u0[:, o:o + 1] + 1e-16)
             + u1[:, :o] / (u1[:, o:o + 1] + 1e-16)) * 0.5
        out_ref[...] = (m + b_ref[0]
                        + jnp.dot(h_ref[...], w_ref[...],
                                  preferred_element_type=jnp.float32)
                        + sb_ref[0])

    return pl.pallas_call(
        body,
        out_shape=jax.ShapeDtypeStruct((N, o), jnp.float32),
        interpret=interpret,
    )(u, h_in, skip_w, bias.reshape(1, o), skip_b.reshape(1, o))


def _head_major(basis, q, k, cin, out_ch):
    wp = 128
    b4 = basis.reshape(NB, cin, H, out_ch).transpose(2, 0, 1, 3)
    pad = ((0, 0), (0, 0), (0, 0), (0, wp - out_ch))
    b4 = jnp.pad(b4, pad)
    q3 = jnp.pad(q.reshape(H, 1, out_ch), ((0, 0), (0, 0), (0, wp - out_ch)))
    k3 = jnp.pad(k.reshape(H, 1, out_ch), ((0, 0), (0, 0), (0, wp - out_ch)))
    return b4, q3, k3


def kernel(x, edge_index, edge_type, basis1, comp1, q1, k1, bias1, skipW1,
           skipb1, gamma1, beta1, basis2, comp2, q2, k2, bias2, skipW2,
           skipb2):
    src = edge_index[0]
    dst = edge_index[1]
    base_idx = edge_type * N
    g0 = (base_idx + src).reshape(NS, NCH, CH)
    d0 = (base_idx + dst).reshape(NS, NCH, CH)
    cat = jnp.stack([jnp.stack([g0, d0], axis=2),
                     jnp.stack([g0 + R * N, d0 + R * N], axis=2)])
    dst3 = dst.reshape(NS, NCH, CH)

    zrow1 = jnp.zeros((ZR, HID + 16), jnp.float32)
    zrow2 = jnp.zeros((ZR, OUT + 16), jnp.float32)

    b41, q31, k31 = _head_major(basis1, q1, k1, IN, HID)
    t1, q1t = _dense_tables(x, b41, comp1, q31, k31, HID)
    u1 = _edge_phase(t1, q1t, cat, dst3, zrow1, HID, qs=True)
    h_ = _post_layer1(u1, x, skipW1, bias1, skipb1, gamma1, beta1)

    b42, q32, k32 = _head_major(basis2, q2, k2, HID, OUT)
    t2, q2t = _dense_tables(h_, b42, comp2, q32, k32, OUT)
    u2 = _edge_phase(t2, q2t, cat, dst3, zrow2, OUT, qs=False)
    return _post_layer2(u2, h_, skipW2, bias2, skipb2)
